# Initial kernel scaffold; baseline (speedup 1.0000x reference)
import jax
import jax.numpy as jnp
from jax.experimental import pallas as pl


def kernel(ego_embeddings, edge_index, edge_weight, W_lin, b_lin, ln_gamma, ln_beta):
    raise NotImplementedError("write your pallas kernel here")


# speedup vs baseline: 2.7916x; 2.7916x over previous
import functools

import jax
import jax.numpy as jnp
from jax import lax
from jax.experimental import pallas as pl
from jax.experimental.pallas import tpu as pltpu
from jax.experimental.pallas import tpu_sc as plsc

_NC = 2
_NS = 16
_C = 256


def _make_sc_segment_sum(N, D, ept):
    n_chunks = ept // _C
    K = _C // 128
    rpt = N // _NS
    mesh = plsc.VectorSubcoreMesh(core_axis_name="c", subcore_axis_name="s")

    @functools.partial(
        pl.kernel,
        mesh=mesh,
        out_type=jax.ShapeDtypeStruct((_NC, N, D), jnp.float32),
        scratch_types=[
            pltpu.VMEM_SHARED((N, D), jnp.float32),
            pltpu.VMEM((K, 128), jnp.int32),
            pltpu.VMEM((K, 128), jnp.int32),
            pltpu.VMEM((_C,), jnp.float32),
            pltpu.VMEM((_C, D), jnp.float32),
            pltpu.SemaphoreType.DMA,
        ],
    )
    def sc_seg_sum(ego, srcm, dstm, w, out, acc, src_v, dst_v, w_v, rows_v, sem):
        c = lax.axis_index("c")
        s = lax.axis_index("s")
        wid = s * _NC + c
        zero16 = jnp.zeros((16,), jnp.float32)

        def zbody(i, carry):
            for r in range(D // 16):
                rows_v[i, pl.ds(r * 16, 16)] = zero16
            return carry

        lax.fori_loop(0, _C, zbody, 0)
        base = s * rpt
        left = rpt
        off = 0
        while left > 0:
            step = min(left, _C)
            pltpu.sync_copy(rows_v.at[pl.ds(0, step)],
                            acc.at[pl.ds(base + off, step)])
            off += step
            left -= step
        plsc.subcore_barrier()

        def chunk(j, carry):
            eb = wid * ept + j * _C
            rb = wid * (ept // 128) + j * K
            pltpu.sync_copy(srcm.at[pl.ds(rb, K)], src_v)
            pltpu.sync_copy(dstm.at[pl.ds(rb, K)], dst_v)
            pltpu.sync_copy(w.at[pl.ds(eb, _C)], w_v)
            cps = [
                pltpu.async_copy(ego.at[src_v.at[k]],
                                 rows_v.at[pl.ds(k * 128, 128)], sem)
                for k in range(K)
            ]
            for cp in cps:
                cp.wait()

            def edge16(ib, icarry):
                wvec = w_v[pl.ds(ib * 16, 16)]
                base_i = ib * 16
                dnums = lax.GatherDimensionNumbers(
                    offset_dims=(), collapsed_slice_dims=(0,),
                    start_index_map=(0,))
                for lane in range(16):
                    wb = lax.gather(
                        wvec, jnp.full((16, 1), lane, jnp.int32), dnums, (1,),
                        mode=lax.GatherScatterMode.PROMISE_IN_BOUNDS)
                    i = base_i + lane
                    for r in range(D // 16):
                        sl = pl.ds(r * 16, 16)
                        rows_v[i, sl] = rows_v[i, sl] * wb
                return icarry

            lax.fori_loop(0, _C // 16, edge16, 0)
            for k in range(K):
                pltpu.sync_copy(rows_v.at[pl.ds(k * 128, 128)],
                                acc.at[dst_v.at[k]], add=True)
            return carry

        lax.fori_loop(0, n_chunks, chunk, 0)
        plsc.subcore_barrier()
        pltpu.sync_copy(acc.at[pl.ds(base, rpt)],
                        out.at[c, pl.ds(base, rpt)])

    return sc_seg_sum


def _tc_dense(ego, p0, p1, W, b, g, beta):
    N, D = ego.shape
    BM = N // 8
    grid = (N // BM,)

    def body(ego_ref, p0_ref, p1_ref, w_ref, b_ref, g_ref, bt_ref, o_ref):
        hi = ego_ref[...] + p0_ref[...] + p1_ref[...]
        e = lax.dot_general(hi, w_ref[...], (((1,), (1,)), ((), ())),
                            preferred_element_type=jnp.float32)
        e = e + b_ref[...]
        e = jnp.where(e >= 0, e, e * 0.01)
        mu = jnp.mean(e, axis=-1, keepdims=True)
        var = jnp.mean((e - mu) ** 2, axis=-1, keepdims=True)
        o_ref[...] = (e - mu) * lax.rsqrt(var + 1e-5) * g_ref[...] + bt_ref[...]

    return pl.pallas_call(
        body,
        grid=grid,
        in_specs=[
            pl.BlockSpec((BM, D), lambda i: (i, 0)),
            pl.BlockSpec((BM, D), lambda i: (i, 0)),
            pl.BlockSpec((BM, D), lambda i: (i, 0)),
            pl.BlockSpec((D, D), lambda i: (0, 0)),
            pl.BlockSpec((1, D), lambda i: (0, 0)),
            pl.BlockSpec((1, D), lambda i: (0, 0)),
            pl.BlockSpec((1, D), lambda i: (0, 0)),
        ],
        out_specs=pl.BlockSpec((BM, D), lambda i: (i, 0)),
        out_shape=jax.ShapeDtypeStruct((N, D), jnp.float32),
    )(ego, p0, p1, W, b.reshape(1, D), g.reshape(1, D), beta.reshape(1, D))


def kernel(ego_embeddings, edge_index, edge_weight, W_lin, b_lin, ln_gamma, ln_beta):
    N, D = ego_embeddings.shape
    E = edge_weight.shape[0]
    NW = _NC * _NS
    ept = -(-E // (NW * _C)) * _C
    E_pad = ept * NW
    pad = E_pad - E
    src = edge_index[0]
    dst = edge_index[1]
    w = edge_weight
    if pad:
        src = jnp.concatenate([src, jnp.zeros((pad,), src.dtype)])
        dst = jnp.concatenate([dst, jnp.zeros((pad,), dst.dtype)])
        w = jnp.concatenate([w, jnp.zeros((pad,), w.dtype)])
    srcm = src.reshape(E_pad // 128, 128)
    dstm = dst.reshape(E_pad // 128, 128)

    N_pad = -(-N // (8 * _NS)) * (8 * _NS)
    sc = _make_sc_segment_sum(N_pad, D, ept)
    partials = sc(ego_embeddings, srcm, dstm, w)
    ego_p = ego_embeddings
    if N_pad != N:
        ego_p = jnp.concatenate(
            [ego_embeddings, jnp.zeros((N_pad - N, D), ego_embeddings.dtype)])
    out = _tc_dense(ego_p, partials[0], partials[1],
                    W_lin, b_lin, ln_gamma, ln_beta)
    return out[:N]


# speedup vs baseline: 3.4179x; 1.2243x over previous
import functools

import jax
import jax.numpy as jnp
from jax import lax
from jax.experimental import pallas as pl
from jax.experimental.pallas import tpu as pltpu
from jax.experimental.pallas import tpu_sc as plsc

_NC = 2
_NS = 16
_C = 128


def _make_sc_segment_sum(N, D, ept):
    nch = ept // _C
    P = nch // 2
    rpt = N // _NS
    mesh = plsc.VectorSubcoreMesh(core_axis_name="c", subcore_axis_name="s")

    @functools.partial(
        pl.kernel,
        mesh=mesh,
        out_type=jax.ShapeDtypeStruct((_NC, N, D), jnp.float32),
        scratch_types=[
            pltpu.VMEM_SHARED((N, D), jnp.float32),
            pltpu.VMEM((2, 128), jnp.int32),
            pltpu.VMEM((2, 128), jnp.int32),
            pltpu.VMEM((_C,), jnp.float32),
            pltpu.VMEM((_C,), jnp.float32),
            pltpu.VMEM((_C, D), jnp.float32),
            pltpu.VMEM((_C, D), jnp.float32),
            pltpu.SemaphoreType.DMA,
            pltpu.SemaphoreType.DMA,
        ],
    )
    def sc_seg_sum(ego, pk, wh, out, acc, idx0, idx1, w0, w1, rows0, rows1,
                   g0, g1):
        c = lax.axis_index("c")
        s = lax.axis_index("s")
        wid = s * _NC + c
        pk_base = wid * nch
        zero16 = jnp.zeros((16,), jnp.float32)

        def scale(w_v, rows_v):
            dnums = lax.GatherDimensionNumbers(
                offset_dims=(), collapsed_slice_dims=(0,),
                start_index_map=(0,))

            def body(ib, carry):
                wvec = w_v[pl.ds(ib * 16, 16)]
                for lane in range(16):
                    wb = lax.gather(
                        wvec, jnp.full((16, 1), lane, jnp.int32), dnums, (1,),
                        mode=lax.GatherScatterMode.PROMISE_IN_BOUNDS)
                    i = ib * 16 + lane
                    for r in range(D // 16):
                        sl = pl.ds(r * 16, 16)
                        rows_v[i, sl] = rows_v[i, sl] * wb
                return carry

            lax.fori_loop(0, _C // 16, body, 0)

        def gather(idx_v, w_v, rows_v, sem, chunk):
            pltpu.sync_copy(pk.at[pk_base + chunk], idx_v)
            pltpu.sync_copy(wh.at[pl.ds((pk_base + chunk) * _C, _C)], w_v)
            pltpu.async_copy(ego.at[idx_v.at[0]], rows_v, sem)

        def wait_gather(idx_v, rows_v, sem):
            pltpu.make_async_copy(ego.at[idx_v.at[0]], rows_v, sem).wait()

        def scatter(idx_v, rows_v):
            pltpu.sync_copy(rows_v, acc.at[idx_v.at[1]], add=True)

        def zbody(i, carry):
            for r in range(D // 16):
                rows0[i, pl.ds(r * 16, 16)] = zero16
            return carry

        lax.fori_loop(0, _C, zbody, 0)
        base = s * rpt
        left = rpt
        off = 0
        while left > 0:
            step = min(left, _C)
            pltpu.sync_copy(rows0.at[pl.ds(0, step)],
                            acc.at[pl.ds(base + off, step)])
            off += step
            left -= step
        plsc.subcore_barrier()

        gather(idx0, w0, rows0, g0, 0)

        def pair(jj, carry):
            gather(idx1, w1, rows1, g1, 2 * jj + 1)
            wait_gather(idx0, rows0, g0)
            scale(w0, rows0)
            scatter(idx0, rows0)
            pl.when(jj < P - 1)(
                lambda: gather(idx0, w0, rows0, g0, 2 * jj + 2))
            wait_gather(idx1, rows1, g1)
            scale(w1, rows1)
            scatter(idx1, rows1)
            return carry

        lax.fori_loop(0, P, pair, 0)

        plsc.subcore_barrier()
        pltpu.sync_copy(acc.at[pl.ds(base, rpt)],
                        out.at[c, pl.ds(base, rpt)])

    return sc_seg_sum


def _tc_dense(ego, p0, p1, W, b, g, beta):
    N, D = ego.shape
    BM = N // 8
    grid = (N // BM,)

    def body(ego_ref, p0_ref, p1_ref, w_ref, b_ref, g_ref, bt_ref, o_ref):
        hi = ego_ref[...] + p0_ref[...] + p1_ref[...]
        e = lax.dot_general(hi, w_ref[...], (((1,), (1,)), ((), ())),
                            preferred_element_type=jnp.float32)
        e = e + b_ref[...]
        e = jnp.where(e >= 0, e, e * 0.01)
        mu = jnp.mean(e, axis=-1, keepdims=True)
        var = jnp.mean((e - mu) ** 2, axis=-1, keepdims=True)
        o_ref[...] = (e - mu) * lax.rsqrt(var + 1e-5) * g_ref[...] + bt_ref[...]

    return pl.pallas_call(
        body,
        grid=grid,
        in_specs=[
            pl.BlockSpec((BM, D), lambda i: (i, 0)),
            pl.BlockSpec((BM, D), lambda i: (i, 0)),
            pl.BlockSpec((BM, D), lambda i: (i, 0)),
            pl.BlockSpec((D, D), lambda i: (0, 0)),
            pl.BlockSpec((1, D), lambda i: (0, 0)),
            pl.BlockSpec((1, D), lambda i: (0, 0)),
            pl.BlockSpec((1, D), lambda i: (0, 0)),
        ],
        out_specs=pl.BlockSpec((BM, D), lambda i: (i, 0)),
        out_shape=jax.ShapeDtypeStruct((N, D), jnp.float32),
    )(ego, p0, p1, W, b.reshape(1, D), g.reshape(1, D), beta.reshape(1, D))


def kernel(ego_embeddings, edge_index, edge_weight, W_lin, b_lin, ln_gamma, ln_beta):
    N, D = ego_embeddings.shape
    E = edge_weight.shape[0]
    NW = _NC * _NS
    ept = -(-E // (NW * 2 * _C)) * (2 * _C)
    E_pad = ept * NW
    pad = E_pad - E
    src = edge_index[0]
    dst = edge_index[1]
    w = edge_weight
    if pad:
        src = jnp.concatenate([src, jnp.zeros((pad,), src.dtype)])
        dst = jnp.concatenate([dst, jnp.zeros((pad,), dst.dtype)])
        w = jnp.concatenate([w, jnp.zeros((pad,), w.dtype)])
    srcm = src.reshape(E_pad // _C, _C)
    dstm = dst.reshape(E_pad // _C, _C)
    pk = jnp.stack([srcm, dstm], axis=1)

    N_pad = -(-N // (8 * _NS)) * (8 * _NS)
    sc = _make_sc_segment_sum(N_pad, D, ept)
    partials = sc(ego_embeddings, pk, w)
    ego_p = ego_embeddings
    if N_pad != N:
        ego_p = jnp.concatenate(
            [ego_embeddings, jnp.zeros((N_pad - N, D), ego_embeddings.dtype)])
    out = _tc_dense(ego_p, partials[0], partials[1],
                    W_lin, b_lin, ln_gamma, ln_beta)
    return out[:N]
